# baseline (speedup 1.0000x reference)
import functools

import jax
import jax.numpy as jnp
from jax import lax
from jax.experimental import pallas as pl
from jax.experimental.pallas import tpu as pltpu
from jax.experimental.pallas import tpu_sc as plsc

N = 10000
E = 320000
D = 128
H = 20
HP = 32
C = 10

NC = 2
NS = 16
NW = NC * NS
CH = 128
NCHUNK = 80
EPT = NCHUNK * CH
EP = NW * EPT
NA = 10240
PR = NA // 4
NBUF = 4
EPR = E // NW
RPT = NA // NS
QR = PR // 4


def _sc_agg_body(ei_hbm, y_hbm, zeros_hbm, out_hbm,
                 src_t, dst1d, dst_t, rows, y_s, agg_s, gsem, ssem):
    c = lax.axis_index("c")
    s = lax.axis_index("s")
    wid = c * NS + s

    pltpu.sync_copy(ei_hbm.at[0, pl.ds(wid * EPR, EPR)],
                    src_t.at[pl.ds(0, EPR)])
    pltpu.sync_copy(ei_hbm.at[1, pl.ds(wid * EPR, EPR)], dst1d)
    chunk = s // 4
    quart = s % 4
    pltpu.sync_copy(
        y_hbm.at[pl.ds(quart * QR, QR), pl.ds(chunk * HP, HP)],
        y_s.at[pl.ds(chunk * PR + quart * QR, QR)])
    pltpu.sync_copy(zeros_hbm.at[pl.ds(s * RPT, RPT)],
                    agg_s.at[0, pl.ds(s * RPT, RPT)])
    pltpu.sync_copy(zeros_hbm.at[pl.ds(s * RPT, RPT)],
                    agg_s.at[1, pl.ds(s * RPT, RPT)])
    rep = s % 2

    zs = jnp.zeros((16,), jnp.int32)
    ds_pad = jnp.full((16,), N, jnp.int32)
    for k in range((EPT - EPR) // 16):
        src_t[pl.ds(EPR + 16 * k, 16)] = zs

    def rp(j, carry):
        for k in range(CH // 16):
            dst_t[j, pl.ds(16 * k, 16)] = dst1d[pl.ds(j * CH + 16 * k, 16)]
        return carry

    full_rows = EPR // CH
    lax.fori_loop(0, full_rows, rp, 0)
    dst_t[full_rows, pl.ds(0, 16)] = dst1d[pl.ds(full_rows * CH, 16)]
    for k in range(1, CH // 16):
        dst_t[full_rows, pl.ds(16 * k, 16)] = ds_pad
    for k in range(CH // 16):
        dst_t[full_rows + 1, pl.ds(16 * k, 16)] = ds_pad
    plsc.subcore_barrier()

    def gather_start(j, b):
        pltpu.async_copy(y_s.at[src_t.at[pl.ds(j * CH, CH)]], rows.at[b],
                         gsem.at[b])

    def gather_wait(j, b):
        pltpu.make_async_copy(y_s.at[src_t.at[pl.ds(j * CH, CH)]], rows.at[b],
                              gsem.at[b]).wait()

    def scatter_start(j, b):
        pltpu.async_copy(rows.at[b], agg_s.at[rep].at[dst_t.at[j]],
                         ssem.at[b], add=True)

    def scatter_wait(j, b):
        pltpu.make_async_copy(rows.at[b], agg_s.at[rep].at[dst_t.at[j]],
                              ssem.at[b]).wait()

    for k in range(NBUF - 1):
        gather_start(k, k)

    def outer(j0, carry):
        for b in range(NBUF):
            j = j0 * NBUF + b
            gather_wait(j, j % NBUF)
            scatter_start(j, j % NBUF)
            if b == 0:
                @pl.when(j0 == 0)
                def _():
                    gather_start(NBUF - 1, NBUF - 1)

                cond = j0 >= 1
            else:
                cond = j0 < NCHUNK // NBUF - 1

            @pl.when(cond)
            def _():
                scatter_wait(j - 1, (j - 1) % NBUF)
                gather_start(j + NBUF - 1, (j - 1) % NBUF)
        return carry

    lax.fori_loop(0, NCHUNK // NBUF, outer, 0)

    for j in range(NCHUNK - NBUF, NCHUNK):
        scatter_wait(j, j % NBUF)

    plsc.subcore_barrier()
    pltpu.sync_copy(
        agg_s.at[0, pl.ds(chunk * PR + quart * QR, QR)],
        out_hbm.at[0, c, pl.ds(quart * QR, QR), pl.ds(chunk * HP, HP)])
    pltpu.sync_copy(
        agg_s.at[1, pl.ds(chunk * PR + quart * QR, QR)],
        out_hbm.at[1, c, pl.ds(quart * QR, QR), pl.ds(chunk * HP, HP)])


_sc_agg = functools.partial(
    pl.kernel,
    out_type=jax.ShapeDtypeStruct((2, NC, PR, 4 * HP), jnp.float32),
    mesh=plsc.VectorSubcoreMesh(core_axis_name="c", subcore_axis_name="s"),
    scratch_types=[
        pltpu.VMEM((EPT,), jnp.int32),
        pltpu.VMEM((EPR,), jnp.int32),
        pltpu.VMEM((NCHUNK, CH), jnp.int32),
        pltpu.VMEM((NBUF, CH, HP), jnp.float32),
        pltpu.VMEM_SHARED((NA, HP), jnp.float32),
        pltpu.VMEM_SHARED((2, NA, HP), jnp.float32),
        pltpu.SemaphoreType.DMA((NBUF,)),
        pltpu.SemaphoreType.DMA((NBUF,)),
    ],
    compiler_params=pltpu.CompilerParams(use_tc_tiling_on_sc=False),
)(_sc_agg_body)


def _proj_body(x_ref, w_ref, o_ref):
    parts = []
    for cc in range(4):
        lo = cc * PR
        hi = min(lo + PR, N)
        yc = jnp.dot(x_ref[lo:hi], w_ref[...],
                     preferred_element_type=jnp.float32)
        if hi - lo < PR:
            yc = jnp.concatenate(
                [yc, jnp.zeros((PR - (hi - lo), HP), jnp.float32)], axis=0)
        parts.append(yc)
    o_ref[...] = jnp.concatenate(parts, axis=1)


_proj = pl.pallas_call(
    _proj_body,
    out_shape=jax.ShapeDtypeStruct((PR, 4 * HP), jnp.float32),
)


def _packed_mlp(p_ref, ba_ref, wbd_ref, bb_ref, sel_ref, selt_ref):
    a = (p_ref[0, 0] + p_ref[0, 1]) + (p_ref[1, 0] + p_ref[1, 1])
    t = jnp.maximum(a + ba_ref[...], 0.0)
    u = jnp.maximum(
        jnp.dot(t, wbd_ref[...], preferred_element_type=jnp.float32)
        + bb_ref[...], 0.0)
    ss = jnp.dot(u * u, sel_ref[...], preferred_element_type=jnp.float32)
    nrm = jnp.maximum(jnp.sqrt(ss), 1e-12)
    den = jnp.dot(nrm, selt_ref[...], preferred_element_type=jnp.float32)
    return jnp.maximum(u / den, 0.0)


def _mid_body(p_ref, ba_ref, wbd_ref, bb_ref, sel_ref, selt_ref, wnd_ref,
              o_ref):
    h = _packed_mlp(p_ref, ba_ref, wbd_ref, bb_ref, sel_ref, selt_ref)
    o_ref[...] = jnp.dot(h, wnd_ref[...], preferred_element_type=jnp.float32)


_mid = pl.pallas_call(
    _mid_body,
    out_shape=jax.ShapeDtypeStruct((PR, 4 * HP), jnp.float32),
)


def _fin_body(p_ref, ba_ref, wbd_ref, bb_ref, sel_ref, selt_ref,
              wl1_ref, wl2_ref, bl_ref, o_ref):
    h = _packed_mlp(p_ref, ba_ref, wbd_ref, bb_ref, sel_ref, selt_ref)
    rid = lax.broadcasted_iota(jnp.int32, (PR, 4 * HP), 0)
    cid = lax.broadcasted_iota(jnp.int32, (PR, 4 * HP), 1)
    valid = jnp.logical_or(cid < 3 * HP, rid < N - 3 * PR)
    hv = jnp.where(valid, h, 0.0)
    m = jnp.max(hv, axis=0, keepdims=True)
    sm = jnp.sum(hv, axis=0, keepdims=True)
    m32 = jnp.maximum(jnp.maximum(m[:, 0:HP], m[:, HP:2 * HP]),
                      jnp.maximum(m[:, 2 * HP:3 * HP], m[:, 3 * HP:4 * HP]))
    s32 = (sm[:, 0:HP] + sm[:, HP:2 * HP] + sm[:, 2 * HP:3 * HP]
           + sm[:, 3 * HP:4 * HP]) * (1.0 / N)
    o_ref[...] = (
        jnp.dot(m32, wl1_ref[...], preferred_element_type=jnp.float32)
        + jnp.dot(s32, wl2_ref[...], preferred_element_type=jnp.float32)
        + bl_ref[...])


_fin = pl.pallas_call(
    _fin_body,
    out_shape=jax.ShapeDtypeStruct((1, C), jnp.float32),
)


def _pad_w(w):
    return jnp.pad(w, ((0, HP - H), (0, HP - H)))


def _blkdiag(w):
    return jnp.kron(jnp.eye(4, dtype=jnp.float32), _pad_w(w))


def _pad_b4(b):
    return jnp.tile(jnp.pad(b, (0, HP - H)), 4).reshape(1, 4 * HP)


def kernel(x, edge_index, W1a, b1a, W1b, b1b, W2a, b2a, W2b, b2b,
           W3a, b3a, W3b, b3b, Wlin, blin):
    zeros = jnp.zeros((NA, HP), jnp.float32)

    W1a_p = jnp.pad(W1a, ((0, 0), (0, HP - H)))
    W1b_d, W2a_d, W2b_d, W3a_d, W3b_d = map(
        _blkdiag, (W1b, W2a, W2b, W3a, W3b))
    b1a_p, b1b_p, b2a_p, b2b_p, b3a_p, b3b_p = map(
        _pad_b4, (b1a, b1b, b2a, b2b, b3a, b3b))
    sel = jnp.kron(jnp.eye(4, dtype=jnp.float32),
                   jnp.ones((HP, 1), jnp.float32))
    selt = sel.T
    wl1 = jnp.pad(Wlin[:H], ((0, HP - H), (0, 0)))
    wl2 = jnp.pad(Wlin[H:], ((0, HP - H), (0, 0)))
    bl = blin.reshape(1, C)

    y1 = _proj(x, W1a_p)
    p1 = _sc_agg(edge_index, y1, zeros)
    y2 = _mid(p1, b1a_p, W1b_d, b1b_p, sel, selt, W2a_d)
    p2 = _sc_agg(edge_index, y2, zeros)
    y3 = _mid(p2, b2a_p, W2b_d, b2b_p, sel, selt, W3a_d)
    p3 = _sc_agg(edge_index, y3, zeros)
    return _fin(p3, b3a_p, W3b_d, b3b_p, sel, selt, wl1, wl2, bl)


# speedup vs baseline: 1.1023x; 1.1023x over previous
import functools

import jax
import jax.numpy as jnp
from jax import lax
from jax.experimental import pallas as pl
from jax.experimental.pallas import tpu as pltpu
from jax.experimental.pallas import tpu_sc as plsc

N = 10000
E = 320000
D = 128
H = 20
HP = 32
C = 10

NC = 2
NS = 16
NW = NC * NS
CH = 128
NCHUNK = 80
EPT = NCHUNK * CH
EP = NW * EPT
NA = 10240
PR = NA // 4
NBUF = 4
EPR = E // NW
RPT = NA // NS
QR = PR // 4


def _sc_agg_body(ei_hbm, y_hbm, zeros_hbm, out_hbm,
                 src_t, dst1d, dst_t, rows, y_s, agg_s, gsem, ssem):
    c = lax.axis_index("c")
    s = lax.axis_index("s")
    wid = c * NS + s

    pltpu.sync_copy(ei_hbm.at[0, pl.ds(wid * EPR, EPR)],
                    src_t.at[pl.ds(0, EPR)])
    pltpu.sync_copy(ei_hbm.at[1, pl.ds(wid * EPR, EPR)], dst1d)
    chunk = s // 4
    quart = s % 4
    pltpu.sync_copy(
        y_hbm.at[pl.ds(quart * QR, QR), pl.ds(chunk * HP, HP)],
        y_s.at[pl.ds(chunk * PR + quart * QR, QR)])
    pltpu.sync_copy(zeros_hbm.at[pl.ds(s * RPT, RPT)],
                    agg_s.at[pl.ds(s * RPT, RPT)])

    zs = jnp.zeros((16,), jnp.int32)
    ds_pad = jnp.full((16,), N, jnp.int32)
    for k in range((EPT - EPR) // 16):
        src_t[pl.ds(EPR + 16 * k, 16)] = zs

    def rp(j, carry):
        for k in range(CH // 16):
            dst_t[j, pl.ds(16 * k, 16)] = dst1d[pl.ds(j * CH + 16 * k, 16)]
        return carry

    full_rows = EPR // CH
    lax.fori_loop(0, full_rows, rp, 0)
    dst_t[full_rows, pl.ds(0, 16)] = dst1d[pl.ds(full_rows * CH, 16)]
    for k in range(1, CH // 16):
        dst_t[full_rows, pl.ds(16 * k, 16)] = ds_pad
    for k in range(CH // 16):
        dst_t[full_rows + 1, pl.ds(16 * k, 16)] = ds_pad
    plsc.subcore_barrier()

    def gather_start(j, b):
        pltpu.async_copy(y_s.at[src_t.at[pl.ds(j * CH, CH)]], rows.at[b],
                         gsem.at[b])

    def gather_wait(j, b):
        pltpu.make_async_copy(y_s.at[src_t.at[pl.ds(j * CH, CH)]], rows.at[b],
                              gsem.at[b]).wait()

    def scatter_start(j, b):
        pltpu.async_copy(rows.at[b], agg_s.at[dst_t.at[j]], ssem.at[b],
                         add=True)

    def scatter_wait(j, b):
        pltpu.make_async_copy(rows.at[b], agg_s.at[dst_t.at[j]],
                              ssem.at[b]).wait()

    for k in range(NBUF - 1):
        gather_start(k, k)

    def outer(j0, carry):
        for b in range(NBUF):
            j = j0 * NBUF + b
            gather_wait(j, j % NBUF)
            scatter_start(j, j % NBUF)
            if b == 0:
                @pl.when(j0 == 0)
                def _():
                    gather_start(NBUF - 1, NBUF - 1)

                cond = j0 >= 1
            else:
                cond = j0 < NCHUNK // NBUF - 1

            @pl.when(cond)
            def _():
                scatter_wait(j - 1, (j - 1) % NBUF)
                gather_start(j + NBUF - 1, (j - 1) % NBUF)
        return carry

    lax.fori_loop(0, NCHUNK // NBUF, outer, 0)

    for j in range(NCHUNK - NBUF, NCHUNK):
        scatter_wait(j, j % NBUF)

    plsc.subcore_barrier()
    pltpu.sync_copy(
        agg_s.at[pl.ds(chunk * PR + quart * QR, QR)],
        out_hbm.at[c, pl.ds(quart * QR, QR), pl.ds(chunk * HP, HP)])


_sc_agg = functools.partial(
    pl.kernel,
    out_type=jax.ShapeDtypeStruct((NC, PR, 4 * HP), jnp.float32),
    mesh=plsc.VectorSubcoreMesh(core_axis_name="c", subcore_axis_name="s"),
    scratch_types=[
        pltpu.VMEM((EPT,), jnp.int32),
        pltpu.VMEM((EPR,), jnp.int32),
        pltpu.VMEM((NCHUNK, CH), jnp.int32),
        pltpu.VMEM((NBUF, CH, HP), jnp.float32),
        pltpu.VMEM_SHARED((NA, HP), jnp.float32),
        pltpu.VMEM_SHARED((NA, HP), jnp.float32),
        pltpu.SemaphoreType.DMA((NBUF,)),
        pltpu.SemaphoreType.DMA((NBUF,)),
    ],
    compiler_params=pltpu.CompilerParams(use_tc_tiling_on_sc=False),
)(_sc_agg_body)


def _proj_body(x_ref, w_ref, o_ref):
    parts = []
    for cc in range(4):
        lo = cc * PR
        hi = min(lo + PR, N)
        yc = jnp.dot(x_ref[lo:hi], w_ref[...],
                     preferred_element_type=jnp.float32)
        if hi - lo < PR:
            yc = jnp.concatenate(
                [yc, jnp.zeros((PR - (hi - lo), HP), jnp.float32)], axis=0)
        parts.append(yc)
    o_ref[...] = jnp.concatenate(parts, axis=1)


_proj = pl.pallas_call(
    _proj_body,
    out_shape=jax.ShapeDtypeStruct((PR, 4 * HP), jnp.float32),
)


def _packed_mlp(p_ref, ba_ref, wbd_ref, bb_ref, sel_ref, selt_ref):
    a = p_ref[0] + p_ref[1]
    t = jnp.maximum(a + ba_ref[...], 0.0)
    u = jnp.maximum(
        jnp.dot(t, wbd_ref[...], preferred_element_type=jnp.float32)
        + bb_ref[...], 0.0)
    ss = jnp.dot(u * u, sel_ref[...], preferred_element_type=jnp.float32)
    nrm = jnp.maximum(jnp.sqrt(ss), 1e-12)
    den = jnp.dot(nrm, selt_ref[...], preferred_element_type=jnp.float32)
    return jnp.maximum(u / den, 0.0)


def _mid_body(p_ref, ba_ref, wbd_ref, bb_ref, sel_ref, selt_ref, wnd_ref,
              o_ref):
    h = _packed_mlp(p_ref, ba_ref, wbd_ref, bb_ref, sel_ref, selt_ref)
    o_ref[...] = jnp.dot(h, wnd_ref[...], preferred_element_type=jnp.float32)


_mid = pl.pallas_call(
    _mid_body,
    out_shape=jax.ShapeDtypeStruct((PR, 4 * HP), jnp.float32),
)


def _fin_body(p_ref, ba_ref, wbd_ref, bb_ref, sel_ref, selt_ref,
              wl1_ref, wl2_ref, bl_ref, o_ref):
    h = _packed_mlp(p_ref, ba_ref, wbd_ref, bb_ref, sel_ref, selt_ref)
    rid = lax.broadcasted_iota(jnp.int32, (PR, 4 * HP), 0)
    cid = lax.broadcasted_iota(jnp.int32, (PR, 4 * HP), 1)
    valid = jnp.logical_or(cid < 3 * HP, rid < N - 3 * PR)
    hv = jnp.where(valid, h, 0.0)
    m = jnp.max(hv, axis=0, keepdims=True)
    sm = jnp.sum(hv, axis=0, keepdims=True)
    m32 = jnp.maximum(jnp.maximum(m[:, 0:HP], m[:, HP:2 * HP]),
                      jnp.maximum(m[:, 2 * HP:3 * HP], m[:, 3 * HP:4 * HP]))
    s32 = (sm[:, 0:HP] + sm[:, HP:2 * HP] + sm[:, 2 * HP:3 * HP]
           + sm[:, 3 * HP:4 * HP]) * (1.0 / N)
    o_ref[...] = (
        jnp.dot(m32, wl1_ref[...], preferred_element_type=jnp.float32)
        + jnp.dot(s32, wl2_ref[...], preferred_element_type=jnp.float32)
        + bl_ref[...])


_fin = pl.pallas_call(
    _fin_body,
    out_shape=jax.ShapeDtypeStruct((1, C), jnp.float32),
)


def _pad_w(w):
    return jnp.pad(w, ((0, HP - H), (0, HP - H)))


def _blkdiag(w):
    return jnp.kron(jnp.eye(4, dtype=jnp.float32), _pad_w(w))


def _pad_b4(b):
    return jnp.tile(jnp.pad(b, (0, HP - H)), 4).reshape(1, 4 * HP)


def kernel(x, edge_index, W1a, b1a, W1b, b1b, W2a, b2a, W2b, b2b,
           W3a, b3a, W3b, b3b, Wlin, blin):
    zeros = jnp.zeros((NA, HP), jnp.float32)

    W1a_p = jnp.pad(W1a, ((0, 0), (0, HP - H)))
    W1b_d, W2a_d, W2b_d, W3a_d, W3b_d = map(
        _blkdiag, (W1b, W2a, W2b, W3a, W3b))
    b1a_p, b1b_p, b2a_p, b2b_p, b3a_p, b3b_p = map(
        _pad_b4, (b1a, b1b, b2a, b2b, b3a, b3b))
    sel = jnp.kron(jnp.eye(4, dtype=jnp.float32),
                   jnp.ones((HP, 1), jnp.float32))
    selt = sel.T
    wl1 = jnp.pad(Wlin[:H], ((0, HP - H), (0, 0)))
    wl2 = jnp.pad(Wlin[H:], ((0, HP - H), (0, 0)))
    bl = blin.reshape(1, C)

    y1 = _proj(x, W1a_p)
    p1 = _sc_agg(edge_index, y1, zeros)
    y2 = _mid(p1, b1a_p, W1b_d, b1b_p, sel, selt, W2a_d)
    p2 = _sc_agg(edge_index, y2, zeros)
    y3 = _mid(p2, b2a_p, W2b_d, b2b_p, sel, selt, W3a_d)
    p3 = _sc_agg(edge_index, y3, zeros)
    return _fin(p3, b3a_p, W3b_d, b3b_p, sel, selt, wl1, wl2, bl)
